# Initial kernel scaffold; baseline (speedup 1.0000x reference)
import jax
import jax.numpy as jnp
from jax.experimental import pallas as pl


def kernel(tokens, W):
    raise NotImplementedError("write your pallas kernel here")


# speedup vs baseline: 1.4590x; 1.4590x over previous
import functools

import jax
import jax.numpy as jnp
from jax import lax
from jax.experimental import pallas as pl
from jax.experimental.pallas import tpu as pltpu
from jax.experimental.pallas import tpu_sc as plsc

_CHUNK = 1024


@functools.lru_cache(maxsize=None)
def _build_gather(vocab, emb, batch, chunk):
    info = plsc.get_sparse_core_info()
    nc, ns = info.num_cores, info.num_subcores
    nw = nc * ns
    assert batch % (nw * chunk) == 0
    b_per_w = batch // nw
    n_chunks = b_per_w // chunk
    mesh = plsc.VectorSubcoreMesh(core_axis_name="c", subcore_axis_name="s")

    @functools.partial(
        pl.kernel,
        mesh=mesh,
        out_type=jax.ShapeDtypeStruct((batch, emb), jnp.float32),
        scratch_types=[
            pltpu.VMEM((chunk,), jnp.int32),
            pltpu.VMEM((chunk, emb), jnp.float32),
            pltpu.SemaphoreType.DMA,
        ],
        compiler_params=pltpu.CompilerParams(use_tc_tiling_on_sc=False),
    )
    def k(table_hbm, idx_hbm, out_hbm, idx_v, rows_v, sem):
        wid = lax.axis_index("s") * nc + lax.axis_index("c")
        base = wid * b_per_w

        def body(j, carry):
            off = base + j * chunk
            pltpu.sync_copy(idx_hbm.at[pl.ds(off, chunk)], idx_v)
            pltpu.async_copy(table_hbm.at[idx_v], rows_v, sem).wait()
            pltpu.sync_copy(rows_v, out_hbm.at[pl.ds(off, chunk)])
            return carry

        lax.fori_loop(0, n_chunks, body, 0)

    return k


def kernel(tokens, W):
    batch = tokens.shape[0] * tokens.shape[1]
    flat = tokens.reshape(batch).astype(jnp.int32)
    out = _build_gather(W.shape[0], W.shape[1], batch, _CHUNK)(W, flat)
    return out.reshape(*tokens.shape, W.shape[1])


# speedup vs baseline: 1.4912x; 1.0220x over previous
import functools

import jax
import jax.numpy as jnp
from jax import lax
from jax.experimental import pallas as pl
from jax.experimental.pallas import tpu as pltpu
from jax.experimental.pallas import tpu_sc as plsc

_CHUNK = 1600


@functools.lru_cache(maxsize=None)
def _build_gather(vocab, emb, batch, chunk):
    info = plsc.get_sparse_core_info()
    nc, ns = info.num_cores, info.num_subcores
    nw = nc * ns
    assert batch % (nw * chunk) == 0
    b_per_w = batch // nw
    n_chunks = b_per_w // chunk
    assert n_chunks % 2 == 0
    mesh = plsc.VectorSubcoreMesh(core_axis_name="c", subcore_axis_name="s")

    @functools.partial(
        pl.kernel,
        mesh=mesh,
        out_type=jax.ShapeDtypeStruct((batch, emb), jnp.float32),
        scratch_types=[
            pltpu.VMEM((2, chunk), jnp.int32),
            pltpu.VMEM((2, chunk, emb), jnp.float32),
            pltpu.SemaphoreType.DMA,
            pltpu.SemaphoreType.DMA,
        ],
        compiler_params=pltpu.CompilerParams(use_tc_tiling_on_sc=False),
    )
    def k(table_hbm, idx_hbm, out_hbm, idx_v, rows_v, gsem, wsem):
        wid = lax.axis_index("s") * nc + lax.axis_index("c")
        base = wid * b_per_w

        def start_gather(j, slot):
            pltpu.sync_copy(
                idx_hbm.at[pl.ds(base + j * chunk, chunk)], idx_v.at[slot]
            )
            pltpu.async_copy(table_hbm.at[idx_v.at[slot]], rows_v.at[slot], gsem)

        def wait_gather(slot):
            pltpu.make_async_copy(
                table_hbm.at[idx_v.at[slot]], rows_v.at[slot], gsem
            ).wait()

        def wait_writeback(j, slot):
            pltpu.make_async_copy(
                rows_v.at[slot], out_hbm.at[pl.ds(base + j * chunk, chunk)], wsem
            ).wait()

        start_gather(0, 0)

        def body(j2, carry):
            for b in range(2):
                j = 2 * j2 + b
                slot, nxt = b, 1 - b
                if b == 0:
                    @pl.when(j2 >= 1)
                    def _():
                        wait_writeback(2 * j2 - 1, nxt)
                else:
                    wait_writeback(j - 1, nxt)
                if b == 0:
                    start_gather(j + 1, nxt)
                else:
                    @pl.when(j2 < (n_chunks // 2) - 1)
                    def _():
                        start_gather(j + 1, nxt)
                wait_gather(slot)
                pltpu.async_copy(
                    rows_v.at[slot],
                    out_hbm.at[pl.ds(base + j * chunk, chunk)],
                    wsem,
                )
            return carry

        lax.fori_loop(0, n_chunks // 2, body, 0)
        wait_writeback(n_chunks - 1, 1)

    return k


def kernel(tokens, W):
    batch = tokens.shape[0] * tokens.shape[1]
    flat = tokens.reshape(batch).astype(jnp.int32)
    out = _build_gather(W.shape[0], W.shape[1], batch, _CHUNK)(W, flat)
    return out.reshape(*tokens.shape, W.shape[1])
